# baseline (speedup 1.0000x reference)
import functools

import jax
import jax.numpy as jnp
from jax import lax
from jax.experimental import pallas as pl
from jax.experimental.pallas import tpu as pltpu
from jax.experimental.pallas import tpu_sc as plsc

BATCH = 16384
EMB = 64
NW = 32
B_PER_W = BATCH // NW
NDB = B_PER_W // 16
SLOG = 7
S = 1 << SLOG
CMAX = (1000000 - 1) >> SLOG


def _fetch(tab_h, ring_v, par, c, sem):
  pltpu.async_copy(tab_h.at[:, pl.ds(pl.multiple_of(c * S, S), S)],
                   ring_v.at[par], sem)


def _wait(tab_h, ring_v, sem):
  pltpu.make_async_copy(tab_h.at[:, pl.ds(0, S)], ring_v.at[0], sem).wait()


def _strip_gather(sidx_v, sperm_v, tab_h, out_h, ring_v, staging_v, sem,
                  rowsem):
  jvecs = [lax.iota(jnp.int32, 16) + 16 * q for q in range(4)]

  vec0 = sidx_v[pl.ds(0, 16)]
  c0 = lax.shift_right_logical(
      jax.lax.squeeze(jax.lax.slice(vec0, (0,), (1,)), (0,)), SLOG)
  _fetch(tab_h, ring_v, 0, c0, sem)
  _wait(tab_h, ring_v, sem)
  spec0 = jnp.minimum(c0 + 1, CMAX)
  _fetch(tab_h, ring_v, 1, spec0, sem)

  def body(g, carry):
    cur_strip, cur_par, pending = carry
    base = g * 16
    idxvec = sidx_v[pl.ds(base, 16)]
    permvec = sperm_v[pl.ds(base, 16)]
    for half in range(2):
      for l in range(8):
        lane = half * 8 + l
        idx = jax.lax.squeeze(jax.lax.slice(idxvec, (lane,), (lane + 1,)), (0,))
        prm = jax.lax.squeeze(jax.lax.slice(permvec, (lane,), (lane + 1,)), (0,))
        c = lax.shift_right_logical(idx, SLOG)
        d = jnp.bitwise_and(idx, S - 1)
        adv = c != cur_strip
        other = 1 - cur_par

        @pl.when(adv)
        def _swap(c=c, other=other):
          _wait(tab_h, ring_v, sem)

        miss = jnp.logical_and(adv, pending != c)

        @pl.when(miss)
        def _refetch(c=c, other=other):
          _fetch(tab_h, ring_v, other, c, sem)
          _wait(tab_h, ring_v, sem)

        @pl.when(adv)
        def _speculate(c=c, cur_par=cur_par):
          _fetch(tab_h, ring_v, cur_par, jnp.minimum(c + 1, CMAX), sem)

        cur_par = jax.lax.select(adv, other, cur_par)
        cur_strip = jax.lax.select(adv, c, cur_strip)
        pending = jax.lax.select(adv, jnp.minimum(c + 1, CMAX), pending)
        sv = jnp.full((16,), cur_par, jnp.int32)
        dv = jnp.full((16,), d, jnp.int32)
        for q in range(4):
          vals = plsc.load_gather(ring_v, [sv, jvecs[q], dv])
          staging_v[l, pl.ds(q * 16, 16)] = vals
        pltpu.async_copy(staging_v.at[pl.ds(l, 1)],
                         out_h.at[pl.ds(prm, 1)], rowsem)
      for l in range(8):
        pltpu.make_async_copy(staging_v.at[pl.ds(0, 1)],
                              out_h.at[pl.ds(0, 1)], rowsem).wait()
    return (cur_strip, cur_par, pending)

  lax.fori_loop(0, NDB, body, (c0, jnp.int32(0), spec0))
  _wait(tab_h, ring_v, sem)


def _gather_body(su_h, pu_h, si_h, pi_h, utab_h, itab_h, uout_h, iout_h,
                 idx_v, prm_v, ring_v, staging_v, sem, rowsem):
  wid = lax.axis_index("s") * 2 + lax.axis_index("c")
  base = wid * B_PER_W
  pltpu.sync_copy(su_h.at[pl.ds(base, B_PER_W)], idx_v)
  pltpu.sync_copy(pu_h.at[pl.ds(base, B_PER_W)], prm_v)
  _strip_gather(idx_v, prm_v, utab_h, uout_h, ring_v, staging_v, sem, rowsem)
  pltpu.sync_copy(si_h.at[pl.ds(base, B_PER_W)], idx_v)
  pltpu.sync_copy(pi_h.at[pl.ds(base, B_PER_W)], prm_v)
  _strip_gather(idx_v, prm_v, itab_h, iout_h, ring_v, staging_v, sem, rowsem)


def _sc_gather(su, pu, si, pi, utab_t, itab_t):
  mesh = plsc.VectorSubcoreMesh(core_axis_name="c", subcore_axis_name="s")
  k = pl.kernel(
      _gather_body,
      out_type=[
          jax.ShapeDtypeStruct((BATCH, EMB), jnp.float32),
          jax.ShapeDtypeStruct((BATCH, EMB), jnp.float32),
      ],
      mesh=mesh,
      scratch_types=[
          pltpu.VMEM((B_PER_W,), jnp.int32),
          pltpu.VMEM((B_PER_W,), jnp.int32),
          pltpu.VMEM((2, EMB, S), jnp.float32),
          pltpu.VMEM((8, EMB), jnp.float32),
          pltpu.SemaphoreType.DMA,
          pltpu.SemaphoreType.DMA,
      ],
      compiler_params=pltpu.CompilerParams(
          needs_layout_passes=False, use_tc_tiling_on_sc=True),
  )
  return k(su, pu, si, pi, utab_t, itab_t)


def _mlp_body(ue_ref, ie_ref, w1_ref, b1_ref, w2_ref, b2_ref, w3_ref, b3_ref,
              out_ref):
  ue = ue_ref[...]
  ie = ie_ref[...]
  w1 = w1_ref[...]
  dn = (((1,), (1,)), ((), ()))
  h = lax.dot_general(ue, w1[:, :EMB], dn, preferred_element_type=jnp.float32)
  h += lax.dot_general(ie, w1[:, EMB:], dn, preferred_element_type=jnp.float32)
  h = jnp.maximum(h + b1_ref[...][None, :], 0.0)
  h2 = lax.dot_general(h, w2_ref[...], dn, preferred_element_type=jnp.float32)
  h2 = jnp.maximum(h2 + b2_ref[...][None, :], 0.0)
  y = jnp.sum(h2 * w3_ref[...], axis=1, keepdims=True)
  out_ref[...] = y + b3_ref[...][None, :]


def _mlp(ue, ie, W1, b1, W2, b2, W3, b3, block=2048):
  nblk = BATCH // block
  return pl.pallas_call(
      _mlp_body,
      grid=(nblk,),
      in_specs=[
          pl.BlockSpec((block, EMB), lambda i: (i, 0)),
          pl.BlockSpec((block, EMB), lambda i: (i, 0)),
          pl.BlockSpec(W1.shape, lambda i: (0, 0)),
          pl.BlockSpec(b1.shape, lambda i: (0,)),
          pl.BlockSpec(W2.shape, lambda i: (0, 0)),
          pl.BlockSpec(b2.shape, lambda i: (0,)),
          pl.BlockSpec(W3.shape, lambda i: (0, 0)),
          pl.BlockSpec(b3.shape, lambda i: (0,)),
      ],
      out_specs=pl.BlockSpec((block, 1), lambda i: (i, 0)),
      out_shape=jax.ShapeDtypeStruct((BATCH, 1), jnp.float32),
      compiler_params=pltpu.CompilerParams(
          dimension_semantics=("parallel",)),
  )(ue, ie, W1, b1, W2, b2, W3, b3)


@jax.jit
def kernel(user, item, user_table, item_table, W1, b1, W2, b2, W3, b3):
  user = user.astype(jnp.int32)
  item = item.astype(jnp.int32)
  pu = jnp.argsort(user).astype(jnp.int32)
  su = jnp.take(user, pu)
  pi = jnp.argsort(item).astype(jnp.int32)
  si = jnp.take(item, pi)
  ue, ie = _sc_gather(su, pu, si, pi, user_table.T, item_table.T)
  y = _mlp(ue, ie, W1, b1, W2, b2, W3, b3)
  return y.reshape(BATCH)


# speedup vs baseline: 1.3474x; 1.3474x over previous
import functools

import jax
import jax.numpy as jnp
from jax import lax
from jax.experimental import pallas as pl
from jax.experimental.pallas import tpu as pltpu
from jax.experimental.pallas import tpu_sc as plsc

BATCH = 16384
EMB = 64
NW = 32
B_PER_W = BATCH // NW
NDB = B_PER_W // 16
SLOG = 8
S = 1 << SLOG
CMAX = (1000000 - 1) >> SLOG


def _fetch(tab_h, ring_v, par, c, sem):
  pltpu.async_copy(tab_h.at[:, pl.ds(pl.multiple_of(c * S, S), S)],
                   ring_v.at[par], sem)


def _wait(tab_h, ring_v, sem):
  pltpu.make_async_copy(tab_h.at[:, pl.ds(0, S)], ring_v.at[0], sem).wait()


def _strip_gather(sidx_v, sperm_v, tab_h, out_h, ring_v, staging_v, sem,
                  rowsem):
  jvecs = [lax.iota(jnp.int32, 16) + 16 * q for q in range(4)]

  vec0 = sidx_v[pl.ds(0, 16)]
  c0 = lax.shift_right_logical(
      jax.lax.squeeze(jax.lax.slice(vec0, (0,), (1,)), (0,)), SLOG)
  _fetch(tab_h, ring_v, 0, c0, sem)
  _wait(tab_h, ring_v, sem)
  spec0 = jnp.minimum(c0 + 1, CMAX)
  _fetch(tab_h, ring_v, 1, spec0, sem)

  def body(g, carry):
    cur_strip, cur_par, pending = carry
    base = g * 16
    idxvec = sidx_v[pl.ds(base, 16)]
    permvec = sperm_v[pl.ds(base, 16)]
    for half in range(2):
      for l in range(8):
        lane = half * 8 + l
        idx = jax.lax.squeeze(jax.lax.slice(idxvec, (lane,), (lane + 1,)), (0,))
        prm = jax.lax.squeeze(jax.lax.slice(permvec, (lane,), (lane + 1,)), (0,))
        c = lax.shift_right_logical(idx, SLOG)
        d = jnp.bitwise_and(idx, S - 1)
        adv = c != cur_strip
        other = 1 - cur_par

        @pl.when(adv)
        def _swap(c=c, other=other):
          _wait(tab_h, ring_v, sem)

        miss = jnp.logical_and(adv, pending != c)

        @pl.when(miss)
        def _refetch(c=c, other=other):
          _fetch(tab_h, ring_v, other, c, sem)
          _wait(tab_h, ring_v, sem)

        @pl.when(adv)
        def _speculate(c=c, cur_par=cur_par):
          _fetch(tab_h, ring_v, cur_par, jnp.minimum(c + 1, CMAX), sem)

        cur_par = jax.lax.select(adv, other, cur_par)
        cur_strip = jax.lax.select(adv, c, cur_strip)
        pending = jax.lax.select(adv, jnp.minimum(c + 1, CMAX), pending)
        sv = jnp.full((16,), cur_par, jnp.int32)
        dv = jnp.full((16,), d, jnp.int32)
        for q in range(4):
          vals = plsc.load_gather(ring_v, [sv, jvecs[q], dv])
          staging_v[l, pl.ds(q * 16, 16)] = vals
        pltpu.async_copy(staging_v.at[pl.ds(l, 1)],
                         out_h.at[pl.ds(prm, 1)], rowsem)
      for l in range(8):
        pltpu.make_async_copy(staging_v.at[pl.ds(0, 1)],
                              out_h.at[pl.ds(0, 1)], rowsem).wait()
    return (cur_strip, cur_par, pending)

  lax.fori_loop(0, NDB, body, (c0, jnp.int32(0), spec0))
  _wait(tab_h, ring_v, sem)


def _gather_body(su_h, pu_h, si_h, pi_h, utab_h, itab_h, uout_h, iout_h,
                 idx_v, prm_v, ring_v, staging_v, sem, rowsem):
  wid = lax.axis_index("s") * 2 + lax.axis_index("c")
  base = wid * B_PER_W
  pltpu.sync_copy(su_h.at[pl.ds(base, B_PER_W)], idx_v)
  pltpu.sync_copy(pu_h.at[pl.ds(base, B_PER_W)], prm_v)
  _strip_gather(idx_v, prm_v, utab_h, uout_h, ring_v, staging_v, sem, rowsem)
  pltpu.sync_copy(si_h.at[pl.ds(base, B_PER_W)], idx_v)
  pltpu.sync_copy(pi_h.at[pl.ds(base, B_PER_W)], prm_v)
  _strip_gather(idx_v, prm_v, itab_h, iout_h, ring_v, staging_v, sem, rowsem)


def _sc_gather(su, pu, si, pi, utab_t, itab_t):
  mesh = plsc.VectorSubcoreMesh(core_axis_name="c", subcore_axis_name="s")
  k = pl.kernel(
      _gather_body,
      out_type=[
          jax.ShapeDtypeStruct((BATCH, EMB), jnp.float32),
          jax.ShapeDtypeStruct((BATCH, EMB), jnp.float32),
      ],
      mesh=mesh,
      scratch_types=[
          pltpu.VMEM((B_PER_W,), jnp.int32),
          pltpu.VMEM((B_PER_W,), jnp.int32),
          pltpu.VMEM((2, EMB, S), jnp.float32),
          pltpu.VMEM((8, EMB), jnp.float32),
          pltpu.SemaphoreType.DMA,
          pltpu.SemaphoreType.DMA,
      ],
      compiler_params=pltpu.CompilerParams(
          needs_layout_passes=False, use_tc_tiling_on_sc=True),
  )
  return k(su, pu, si, pi, utab_t, itab_t)


def _mlp_body(ue_ref, ie_ref, w1_ref, b1_ref, w2_ref, b2_ref, w3_ref, b3_ref,
              out_ref):
  ue = ue_ref[...]
  ie = ie_ref[...]
  w1 = w1_ref[...]
  dn = (((1,), (1,)), ((), ()))
  h = lax.dot_general(ue, w1[:, :EMB], dn, preferred_element_type=jnp.float32)
  h += lax.dot_general(ie, w1[:, EMB:], dn, preferred_element_type=jnp.float32)
  h = jnp.maximum(h + b1_ref[...][None, :], 0.0)
  h2 = lax.dot_general(h, w2_ref[...], dn, preferred_element_type=jnp.float32)
  h2 = jnp.maximum(h2 + b2_ref[...][None, :], 0.0)
  y = jnp.sum(h2 * w3_ref[...], axis=1, keepdims=True)
  out_ref[...] = y + b3_ref[...][None, :]


def _mlp(ue, ie, W1, b1, W2, b2, W3, b3, block=2048):
  nblk = BATCH // block
  return pl.pallas_call(
      _mlp_body,
      grid=(nblk,),
      in_specs=[
          pl.BlockSpec((block, EMB), lambda i: (i, 0)),
          pl.BlockSpec((block, EMB), lambda i: (i, 0)),
          pl.BlockSpec(W1.shape, lambda i: (0, 0)),
          pl.BlockSpec(b1.shape, lambda i: (0,)),
          pl.BlockSpec(W2.shape, lambda i: (0, 0)),
          pl.BlockSpec(b2.shape, lambda i: (0,)),
          pl.BlockSpec(W3.shape, lambda i: (0, 0)),
          pl.BlockSpec(b3.shape, lambda i: (0,)),
      ],
      out_specs=pl.BlockSpec((block, 1), lambda i: (i, 0)),
      out_shape=jax.ShapeDtypeStruct((BATCH, 1), jnp.float32),
      compiler_params=pltpu.CompilerParams(
          dimension_semantics=("parallel",)),
  )(ue, ie, W1, b1, W2, b2, W3, b3)


@jax.jit
def kernel(user, item, user_table, item_table, W1, b1, W2, b2, W3, b3):
  user = user.astype(jnp.int32)
  item = item.astype(jnp.int32)
  pu = jnp.argsort(user).astype(jnp.int32)
  su = jnp.take(user, pu)
  pi = jnp.argsort(item).astype(jnp.int32)
  si = jnp.take(item, pi)
  ue, ie = _sc_gather(su, pu, si, pi, user_table.T, item_table.T)
  y = _mlp(ue, ie, W1, b1, W2, b2, W3, b3)
  return y.reshape(BATCH)


# speedup vs baseline: 1.7022x; 1.2633x over previous
import functools

import jax
import jax.numpy as jnp
from jax import lax
from jax.experimental import pallas as pl
from jax.experimental.pallas import tpu as pltpu
from jax.experimental.pallas import tpu_sc as plsc

BATCH = 16384
EMB = 64
NW = 32
B_PER_W = BATCH // NW
NDB = B_PER_W // 16
SLOG = 9
S = 1 << SLOG
CMAX = (1000000 - 1) >> SLOG


def _fetch(tab_h, ring_v, par, c, sem):
  pltpu.async_copy(tab_h.at[:, pl.ds(pl.multiple_of(c * S, S), S)],
                   ring_v.at[par], sem)


def _wait(tab_h, ring_v, sem):
  pltpu.make_async_copy(tab_h.at[:, pl.ds(0, S)], ring_v.at[0], sem).wait()


def _strip_gather(sidx_v, sperm_v, tab_h, out_h, ring_v, staging_v, sem,
                  rowsem):
  jvecs = [lax.iota(jnp.int32, 16) + 16 * q for q in range(4)]

  vec0 = sidx_v[pl.ds(0, 16)]
  c0 = lax.shift_right_logical(
      jax.lax.squeeze(jax.lax.slice(vec0, (0,), (1,)), (0,)), SLOG)
  _fetch(tab_h, ring_v, 0, c0, sem)
  _wait(tab_h, ring_v, sem)
  spec0 = jnp.minimum(c0 + 1, CMAX)
  _fetch(tab_h, ring_v, 1, spec0, sem)

  def body(g, carry):
    cur_strip, cur_par, pending = carry
    base = g * 16
    idxvec = sidx_v[pl.ds(base, 16)]
    permvec = sperm_v[pl.ds(base, 16)]
    for half in range(2):
      for l in range(8):
        lane = half * 8 + l
        idx = jax.lax.squeeze(jax.lax.slice(idxvec, (lane,), (lane + 1,)), (0,))
        prm = jax.lax.squeeze(jax.lax.slice(permvec, (lane,), (lane + 1,)), (0,))
        c = lax.shift_right_logical(idx, SLOG)
        d = jnp.bitwise_and(idx, S - 1)
        adv = c != cur_strip
        other = 1 - cur_par

        @pl.when(adv)
        def _swap(c=c, other=other):
          _wait(tab_h, ring_v, sem)

        miss = jnp.logical_and(adv, pending != c)

        @pl.when(miss)
        def _refetch(c=c, other=other):
          _fetch(tab_h, ring_v, other, c, sem)
          _wait(tab_h, ring_v, sem)

        @pl.when(adv)
        def _speculate(c=c, cur_par=cur_par):
          _fetch(tab_h, ring_v, cur_par, jnp.minimum(c + 1, CMAX), sem)

        cur_par = jax.lax.select(adv, other, cur_par)
        cur_strip = jax.lax.select(adv, c, cur_strip)
        pending = jax.lax.select(adv, jnp.minimum(c + 1, CMAX), pending)
        sv = jnp.full((16,), cur_par, jnp.int32)
        dv = jnp.full((16,), d, jnp.int32)
        for q in range(4):
          vals = plsc.load_gather(ring_v, [sv, jvecs[q], dv])
          staging_v[l, pl.ds(q * 16, 16)] = vals
        pltpu.async_copy(staging_v.at[pl.ds(l, 1)],
                         out_h.at[pl.ds(prm, 1)], rowsem)
      for l in range(8):
        pltpu.make_async_copy(staging_v.at[pl.ds(0, 1)],
                              out_h.at[pl.ds(0, 1)], rowsem).wait()
    return (cur_strip, cur_par, pending)

  lax.fori_loop(0, NDB, body, (c0, jnp.int32(0), spec0))
  _wait(tab_h, ring_v, sem)


def _gather_body(su_h, pu_h, si_h, pi_h, utab_h, itab_h, uout_h, iout_h,
                 idx_v, prm_v, ring_v, staging_v, sem, rowsem):
  wid = lax.axis_index("s") * 2 + lax.axis_index("c")
  base = wid * B_PER_W
  pltpu.sync_copy(su_h.at[pl.ds(base, B_PER_W)], idx_v)
  pltpu.sync_copy(pu_h.at[pl.ds(base, B_PER_W)], prm_v)
  _strip_gather(idx_v, prm_v, utab_h, uout_h, ring_v, staging_v, sem, rowsem)
  pltpu.sync_copy(si_h.at[pl.ds(base, B_PER_W)], idx_v)
  pltpu.sync_copy(pi_h.at[pl.ds(base, B_PER_W)], prm_v)
  _strip_gather(idx_v, prm_v, itab_h, iout_h, ring_v, staging_v, sem, rowsem)


def _sc_gather(su, pu, si, pi, utab_t, itab_t):
  mesh = plsc.VectorSubcoreMesh(core_axis_name="c", subcore_axis_name="s")
  k = pl.kernel(
      _gather_body,
      out_type=[
          jax.ShapeDtypeStruct((BATCH, EMB), jnp.float32),
          jax.ShapeDtypeStruct((BATCH, EMB), jnp.float32),
      ],
      mesh=mesh,
      scratch_types=[
          pltpu.VMEM((B_PER_W,), jnp.int32),
          pltpu.VMEM((B_PER_W,), jnp.int32),
          pltpu.VMEM((2, EMB, S), jnp.float32),
          pltpu.VMEM((8, EMB), jnp.float32),
          pltpu.SemaphoreType.DMA,
          pltpu.SemaphoreType.DMA,
      ],
      compiler_params=pltpu.CompilerParams(
          needs_layout_passes=False, use_tc_tiling_on_sc=True),
  )
  return k(su, pu, si, pi, utab_t, itab_t)


def _mlp_body(ue_ref, ie_ref, w1_ref, b1_ref, w2_ref, b2_ref, w3_ref, b3_ref,
              out_ref):
  ue = ue_ref[...]
  ie = ie_ref[...]
  w1 = w1_ref[...]
  dn = (((1,), (1,)), ((), ()))
  h = lax.dot_general(ue, w1[:, :EMB], dn, preferred_element_type=jnp.float32)
  h += lax.dot_general(ie, w1[:, EMB:], dn, preferred_element_type=jnp.float32)
  h = jnp.maximum(h + b1_ref[...][None, :], 0.0)
  h2 = lax.dot_general(h, w2_ref[...], dn, preferred_element_type=jnp.float32)
  h2 = jnp.maximum(h2 + b2_ref[...][None, :], 0.0)
  y = jnp.sum(h2 * w3_ref[...], axis=1, keepdims=True)
  out_ref[...] = y + b3_ref[...][None, :]


def _mlp(ue, ie, W1, b1, W2, b2, W3, b3, block=2048):
  nblk = BATCH // block
  return pl.pallas_call(
      _mlp_body,
      grid=(nblk,),
      in_specs=[
          pl.BlockSpec((block, EMB), lambda i: (i, 0)),
          pl.BlockSpec((block, EMB), lambda i: (i, 0)),
          pl.BlockSpec(W1.shape, lambda i: (0, 0)),
          pl.BlockSpec(b1.shape, lambda i: (0,)),
          pl.BlockSpec(W2.shape, lambda i: (0, 0)),
          pl.BlockSpec(b2.shape, lambda i: (0,)),
          pl.BlockSpec(W3.shape, lambda i: (0, 0)),
          pl.BlockSpec(b3.shape, lambda i: (0,)),
      ],
      out_specs=pl.BlockSpec((block, 1), lambda i: (i, 0)),
      out_shape=jax.ShapeDtypeStruct((BATCH, 1), jnp.float32),
      compiler_params=pltpu.CompilerParams(
          dimension_semantics=("parallel",)),
  )(ue, ie, W1, b1, W2, b2, W3, b3)


@jax.jit
def kernel(user, item, user_table, item_table, W1, b1, W2, b2, W3, b3):
  user = user.astype(jnp.int32)
  item = item.astype(jnp.int32)
  pu = jnp.argsort(user).astype(jnp.int32)
  su = jnp.take(user, pu)
  pi = jnp.argsort(item).astype(jnp.int32)
  si = jnp.take(item, pi)
  ue, ie = _sc_gather(su, pu, si, pi, user_table.T, item_table.T)
  y = _mlp(ue, ie, W1, b1, W2, b2, W3, b3)
  return y.reshape(BATCH)
